# baseline (speedup 1.0000x reference)
import jax
import jax.numpy as jnp
from jax.experimental import pallas as pl
from jax.experimental.pallas import tpu as pltpu

NTOK = 32768
HIDDEN = 4096
NUM_EXPERTS = 64
TOP_K = 8
BLK = 1024


def _router_block(x_ref, w_ref, probs_ref, idx_ref):
    x = x_ref[...]
    w = w_ref[...]
    logits = jax.lax.dot_general(
        x, w, (((1,), (1,)), ((), ())),
        preferred_element_type=jnp.float32,
        precision=jax.lax.Precision.DEFAULT,
    )
    lt = jnp.transpose(logits)
    m = jnp.max(lt, axis=0, keepdims=True)
    e = jnp.exp(lt - m)
    p = e / jnp.sum(e, axis=0, keepdims=True)
    probs_ref[...] = jnp.transpose(p)

    iota_e = jax.lax.broadcasted_iota(jnp.int32, (NUM_EXPERTS, BLK), 0)
    irev = (63 - iota_e).astype(jnp.float32)
    work = p
    rows = []
    for _ in range(TOP_K):
        mx = jnp.max(work, axis=0, keepdims=True)
        sel = jnp.max(jnp.where(work == mx, irev, -1.0),
                      axis=0, keepdims=True)
        amax = 63 - sel.astype(jnp.int32)
        rows.append(amax)
        work = jnp.where(iota_e == amax, -1.0, work)
    idx_ref[...] = jnp.transpose(jnp.concatenate(rows, axis=0))


def kernel(x, W):
    grid = (NTOK // BLK,)
    probs, idx = pl.pallas_call(
        _router_block,
        grid=grid,
        in_specs=[
            pl.BlockSpec((BLK, HIDDEN), lambda i: (i, 0)),
            pl.BlockSpec((NUM_EXPERTS, HIDDEN), lambda i: (0, 0)),
        ],
        out_specs=[
            pl.BlockSpec((BLK, NUM_EXPERTS), lambda i: (i, 0)),
            pl.BlockSpec((BLK, TOP_K), lambda i: (i, 0)),
        ],
        out_shape=[
            jax.ShapeDtypeStruct((NTOK, NUM_EXPERTS), jnp.float32),
            jax.ShapeDtypeStruct((NTOK, TOP_K), jnp.int32),
        ],
        compiler_params=pltpu.CompilerParams(
            dimension_semantics=("parallel",),
        ),
    )(x, W)
    return (probs, idx)
